# baseline (speedup 1.0000x reference)
import math

import jax
import jax.numpy as jnp
from jax.experimental import pallas as pl
from jax.experimental.pallas import tpu as pltpu

T, B, D, H, A = 2048, 16, 128, 128, 32
C = 64
L = T // C
CB = C * B
TILE = 128
TB = 512

_HALF_LOG_2PI = 0.5 * math.log(2.0 * math.pi)
_PREC = jax.lax.Precision.DEFAULT


def _scan_kernel(obs_ref, mask_ref, h0_ref, wih_ref, whh_ref, bih_ref,
                 bhh_ref, n_ref, r_ref, nmax_ref, outs_ref, h_all_s, h2a_s):
    h_all_s[...] = jnp.zeros((CB, H), jnp.float32)
    h_all_s[0:B, :] = h0_ref[...]
    wih = wih_ref[...]
    whh = whh_ref[...]
    bih = bih_ref[...]
    bhh = bhh_ref[...]

    def gru_step(x, h, m):
        hm = h * m
        gi = jax.lax.dot_general(x, wih, (((1,), (1,)), ((), ())),
                                 preferred_element_type=jnp.float32,
                                 precision=_PREC) + bih
        gh = jax.lax.dot_general(hm, whh, (((1,), (1,)), ((), ())),
                                 preferred_element_type=jnp.float32,
                                 precision=_PREC) + bhh
        r = 0.5 * jnp.tanh(0.5 * (gi[:, :H] + gh[:, :H])) + 0.5
        z = 0.5 * jnp.tanh(0.5 * (gi[:, H:2 * H] + gh[:, H:2 * H])) + 0.5
        n = jnp.tanh(gi[:, 2 * H:] + r * gh[:, 2 * H:])
        return n + z * (hm - n)

    def make_batched_step(h_ref):
        def batched_step(s, carry):
            for k in range(CB // TILE):
                ck = k * (TILE // B)
                x = obs_ref[pl.ds(ck, TILE // B), pl.ds(s, 1)].reshape(TILE, D)
                m = mask_ref[pl.ds(ck, TILE // B), pl.ds(s, 1)].reshape(
                    TILE, 1).astype(jnp.float32)
                h = h_ref[pl.ds(k * TILE, TILE), :]
                h_new = gru_step(x, h, m)
                h_ref[pl.ds(k * TILE, TILE), :] = h_new
                outs_ref[pl.ds(ck, TILE // B), pl.ds(s, 1)] = h_new.reshape(
                    TILE // B, 1, B, H).astype(jnp.bfloat16)
            return carry
        return batched_step

    jax.lax.fori_loop(0, L, make_batched_step(h_all_s), 0, unroll=8)

    h2a_s[B:CB, :] = h_all_s[0:CB - B, :]
    h2a_s[0:B, :] = h0_ref[...]
    jax.lax.fori_loop(0, nmax_ref[0], make_batched_step(h2a_s), 0,
                      unroll=False)

    def chunk_body(c, h):
        nc = n_ref[c]
        rc = r_ref[c]

        def s_body(s, h):
            x = obs_ref[pl.ds(c, 1), pl.ds(s, 1)].reshape(B, D)
            m = mask_ref[pl.ds(c, 1), pl.ds(s, 1)].reshape(
                B, 1).astype(jnp.float32)
            h_new = gru_step(x, h, m)
            outs_ref[pl.ds(c, 1), pl.ds(s, 1)] = h_new.reshape(
                1, 1, B, H).astype(jnp.bfloat16)
            return h_new

        h2 = jax.lax.fori_loop(0, rc, s_body, h)
        row = pl.multiple_of(c * B, B)
        he1 = h_all_s[pl.ds(row, B), :]
        h2a_end = h2a_s[pl.ds(row, B), :]
        wb = jnp.where(rc > 0, 1.0, 0.0).astype(jnp.float32)
        wf = jnp.where(nc == L, 1.0, 0.0).astype(jnp.float32)
        h_full = wb * h2 + (1.0 - wb) * h2a_end
        return wf * h_full + (1.0 - wf) * he1

    @pl.when(nmax_ref[1] > 0)
    def _repair():
        jax.lax.fori_loop(1, C, chunk_body, h_all_s[0:B, :])


def _head_kernel(outs_ref, act_ref, wout_ref, bout_ref, ls_ref,
                 lp_ref, ent_ref):
    o = outs_ref[...].reshape(TB * B, H)
    mean = jax.lax.dot_general(o, wout_ref[...], (((1,), (1,)), ((), ())),
                               preferred_element_type=jnp.float32,
                               precision=jax.lax.Precision.DEFAULT) + bout_ref[...]
    a = act_ref[...].reshape(TB * B, A)
    ls = ls_ref[...]
    inv2var = 0.5 * jnp.exp(-2.0 * ls)
    terms = -((a - mean) ** 2) * inv2var - ls - _HALF_LOG_2PI
    lp_ref[...] = jnp.sum(terms, axis=1, keepdims=True)
    ent_ref[...] = jnp.full((TB * B, 1),
                            jnp.sum(0.5 + _HALF_LOG_2PI + ls), jnp.float32)


@jax.jit
def _run(obs, hidden_states, dones, action, W_ih, W_hh, b_ih, b_hh,
         W_out, b_out, log_std):
    obs4 = obs.reshape(C, L, B, D)
    d2 = dones.reshape(C, L, B)
    mask4 = (1.0 - d2).reshape(C, L, B, 1).astype(jnp.bfloat16)
    act3 = action.reshape(T, B, A)
    h0 = hidden_states.reshape(B, H)
    bih = b_ih.reshape(1, 3 * H)
    bhh = b_hh.reshape(1, 3 * H)
    bout = b_out.reshape(1, A)
    ls = log_std.reshape(1, A)

    iota_l = jax.lax.broadcasted_iota(jnp.int32, (C, L, B), 1)
    first = jnp.min(jnp.where(d2 > 0.5, iota_l, L), axis=1)
    n = first.max(axis=1).astype(jnp.int32)
    bad = jnp.concatenate([jnp.zeros((1,), jnp.bool_), n[:-1] == L])
    n = n.at[0].set(0)
    r = jnp.where(bad, n, 0).astype(jnp.int32)
    nmax = jnp.stack([jnp.max(n), jnp.sum(r)])

    outs4 = pl.pallas_call(
        _scan_kernel,
        grid=(1,),
        in_specs=[
            pl.BlockSpec((C, L, B, D), lambda i: (0, 0, 0, 0)),
            pl.BlockSpec((C, L, B, 1), lambda i: (0, 0, 0, 0)),
            pl.BlockSpec((B, H), lambda i: (0, 0)),
            pl.BlockSpec((3 * H, D), lambda i: (0, 0)),
            pl.BlockSpec((3 * H, H), lambda i: (0, 0)),
            pl.BlockSpec((1, 3 * H), lambda i: (0, 0)),
            pl.BlockSpec((1, 3 * H), lambda i: (0, 0)),
            pl.BlockSpec(memory_space=pltpu.SMEM),
            pl.BlockSpec(memory_space=pltpu.SMEM),
            pl.BlockSpec(memory_space=pltpu.SMEM),
        ],
        out_specs=pl.BlockSpec((C, L, B, H), lambda i: (0, 0, 0, 0)),
        out_shape=jax.ShapeDtypeStruct((C, L, B, H), jnp.bfloat16),
        scratch_shapes=[pltpu.VMEM((CB, H), jnp.float32),
                        pltpu.VMEM((CB, H), jnp.float32)],
        compiler_params=pltpu.CompilerParams(
            dimension_semantics=("arbitrary",)),
    )(obs4, mask4, h0, W_ih, W_hh, bih, bhh, n, r, nmax)

    outs3 = outs4.reshape(T, B, H)
    lp, ent = pl.pallas_call(
        _head_kernel,
        grid=(T // TB,),
        in_specs=[
            pl.BlockSpec((TB, B, H), lambda i: (i, 0, 0)),
            pl.BlockSpec((TB, B, A), lambda i: (i, 0, 0)),
            pl.BlockSpec((A, H), lambda i: (0, 0)),
            pl.BlockSpec((1, A), lambda i: (0, 0)),
            pl.BlockSpec((1, A), lambda i: (0, 0)),
        ],
        out_specs=[
            pl.BlockSpec((TB * B, 1), lambda i: (i, 0)),
            pl.BlockSpec((TB * B, 1), lambda i: (i, 0)),
        ],
        out_shape=[
            jax.ShapeDtypeStruct((T * B, 1), jnp.float32),
            jax.ShapeDtypeStruct((T * B, 1), jnp.float32),
        ],
    )(outs3, act3, W_out.astype(jnp.bfloat16), bout, ls)

    return action, lp.reshape(T * B), ent.reshape(T * B)


def kernel(obs, hidden_states, dones, action, W_ih, W_hh, b_ih, b_hh,
           W_out, b_out, log_std):
    return _run(obs, hidden_states, dones, action, W_ih, W_hh, b_ih, b_hh,
                W_out, b_out, log_std)


# speedup vs baseline: 1.0441x; 1.0441x over previous
import math

import jax
import jax.numpy as jnp
from jax.experimental import pallas as pl
from jax.experimental.pallas import tpu as pltpu

T, B, D, H, A = 2048, 16, 128, 128, 32
C = 64
L = T // C
CB = C * B
TILE = 128
TB = 512

_HALF_LOG_2PI = 0.5 * math.log(2.0 * math.pi)
_PREC = jax.lax.Precision.DEFAULT


def _scan_kernel(obs_ref, mask_ref, h0_ref, wih_ref, whh_ref, bih_ref,
                 bhh_ref, n_ref, r_ref, nmax_ref, outs_ref, h_all_s, h2a_s):
    h_all_s[...] = jnp.zeros((CB, H), jnp.float32)
    h_all_s[0:B, :] = h0_ref[...]
    wih = wih_ref[...]
    whh = whh_ref[...]
    bih = bih_ref[...]
    bhh = bhh_ref[...]

    def gru_step(x, h, m):
        hm = h * m
        gi = jax.lax.dot_general(x, wih, (((1,), (0,)), ((), ())),
                                 preferred_element_type=jnp.float32,
                                 precision=_PREC) + bih
        gh = jax.lax.dot_general(hm, whh, (((1,), (0,)), ((), ())),
                                 preferred_element_type=jnp.float32,
                                 precision=_PREC) + bhh
        r = 0.5 * jnp.tanh(0.5 * (gi[:, :H] + gh[:, :H])) + 0.5
        z = 0.5 * jnp.tanh(0.5 * (gi[:, H:2 * H] + gh[:, H:2 * H])) + 0.5
        n = jnp.tanh(gi[:, 2 * H:] + r * gh[:, 2 * H:])
        return n + z * (hm - n)

    def make_batched_step(h_ref):
        def batched_step(s, carry):
            for k in range(CB // TILE):
                ck = k * (TILE // B)
                x = obs_ref[pl.ds(ck, TILE // B), pl.ds(s, 1)].reshape(TILE, D)
                m = mask_ref[pl.ds(ck, TILE // B), pl.ds(s, 1)].reshape(
                    TILE, 1).astype(jnp.float32)
                h = h_ref[pl.ds(k * TILE, TILE), :]
                h_new = gru_step(x, h, m)
                h_ref[pl.ds(k * TILE, TILE), :] = h_new
                outs_ref[pl.ds(ck, TILE // B), pl.ds(s, 1)] = h_new.reshape(
                    TILE // B, 1, B, H).astype(jnp.bfloat16)
            return carry
        return batched_step

    jax.lax.fori_loop(0, L, make_batched_step(h_all_s), 0, unroll=8)

    h2a_s[B:CB, :] = h_all_s[0:CB - B, :]
    h2a_s[0:B, :] = h0_ref[...]
    jax.lax.fori_loop(0, nmax_ref[0], make_batched_step(h2a_s), 0,
                      unroll=False)

    def chunk_body(c, h):
        nc = n_ref[c]
        rc = r_ref[c]

        def s_body(s, h):
            x = obs_ref[pl.ds(c, 1), pl.ds(s, 1)].reshape(B, D)
            m = mask_ref[pl.ds(c, 1), pl.ds(s, 1)].reshape(
                B, 1).astype(jnp.float32)
            h_new = gru_step(x, h, m)
            outs_ref[pl.ds(c, 1), pl.ds(s, 1)] = h_new.reshape(
                1, 1, B, H).astype(jnp.bfloat16)
            return h_new

        h2 = jax.lax.fori_loop(0, rc, s_body, h)
        row = pl.multiple_of(c * B, B)
        he1 = h_all_s[pl.ds(row, B), :]
        h2a_end = h2a_s[pl.ds(row, B), :]
        wb = jnp.where(rc > 0, 1.0, 0.0).astype(jnp.float32)
        wf = jnp.where(nc == L, 1.0, 0.0).astype(jnp.float32)
        h_full = wb * h2 + (1.0 - wb) * h2a_end
        return wf * h_full + (1.0 - wf) * he1

    @pl.when(nmax_ref[1] > 0)
    def _repair():
        jax.lax.fori_loop(1, C, chunk_body, h_all_s[0:B, :])


def _head_kernel(outs_ref, act_ref, wout_ref, bout_ref, ls_ref,
                 lp_ref, ent_ref):
    o = outs_ref[...].reshape(TB * B, H)
    mean = jax.lax.dot_general(o, wout_ref[...], (((1,), (0,)), ((), ())),
                               preferred_element_type=jnp.float32,
                               precision=jax.lax.Precision.DEFAULT) + bout_ref[...]
    a = act_ref[...].reshape(TB * B, A)
    ls = ls_ref[...]
    inv2var = 0.5 * jnp.exp(-2.0 * ls)
    terms = -((a - mean) ** 2) * inv2var - ls - _HALF_LOG_2PI
    lp_ref[...] = jnp.sum(terms, axis=1, keepdims=True)
    ent_ref[...] = jnp.full((TB * B, 1),
                            jnp.sum(0.5 + _HALF_LOG_2PI + ls), jnp.float32)


@jax.jit
def _run(obs, hidden_states, dones, action, W_ih, W_hh, b_ih, b_hh,
         W_out, b_out, log_std):
    obs4 = obs.reshape(C, L, B, D)
    wihT = W_ih.T
    whhT = W_hh.T
    woutT = W_out.T
    d2 = dones.reshape(C, L, B)
    mask4 = (1.0 - d2).reshape(C, L, B, 1).astype(jnp.bfloat16)
    act3 = action.reshape(T, B, A)
    h0 = hidden_states.reshape(B, H)
    bih = b_ih.reshape(1, 3 * H)
    bhh = b_hh.reshape(1, 3 * H)
    bout = b_out.reshape(1, A)
    ls = log_std.reshape(1, A)

    iota_l = jax.lax.broadcasted_iota(jnp.int32, (C, L, B), 1)
    first = jnp.min(jnp.where(d2 > 0.5, iota_l, L), axis=1)
    n = first.max(axis=1).astype(jnp.int32)
    bad = jnp.concatenate([jnp.zeros((1,), jnp.bool_), n[:-1] == L])
    n = n.at[0].set(0)
    r = jnp.where(bad, n, 0).astype(jnp.int32)
    nmax = jnp.stack([jnp.max(n), jnp.sum(r)])

    outs4 = pl.pallas_call(
        _scan_kernel,
        grid=(1,),
        in_specs=[
            pl.BlockSpec((C, L, B, D), lambda i: (0, 0, 0, 0)),
            pl.BlockSpec((C, L, B, 1), lambda i: (0, 0, 0, 0)),
            pl.BlockSpec((B, H), lambda i: (0, 0)),
            pl.BlockSpec((D, 3 * H), lambda i: (0, 0)),
            pl.BlockSpec((H, 3 * H), lambda i: (0, 0)),
            pl.BlockSpec((1, 3 * H), lambda i: (0, 0)),
            pl.BlockSpec((1, 3 * H), lambda i: (0, 0)),
            pl.BlockSpec(memory_space=pltpu.SMEM),
            pl.BlockSpec(memory_space=pltpu.SMEM),
            pl.BlockSpec(memory_space=pltpu.SMEM),
        ],
        out_specs=pl.BlockSpec((C, L, B, H), lambda i: (0, 0, 0, 0)),
        out_shape=jax.ShapeDtypeStruct((C, L, B, H), jnp.bfloat16),
        scratch_shapes=[pltpu.VMEM((CB, H), jnp.float32),
                        pltpu.VMEM((CB, H), jnp.float32)],
        compiler_params=pltpu.CompilerParams(
            dimension_semantics=("arbitrary",)),
    )(obs4, mask4, h0, wihT, whhT, bih, bhh, n, r, nmax)

    outs3 = outs4.reshape(T, B, H)
    lp, ent = pl.pallas_call(
        _head_kernel,
        grid=(T // TB,),
        in_specs=[
            pl.BlockSpec((TB, B, H), lambda i: (i, 0, 0)),
            pl.BlockSpec((TB, B, A), lambda i: (i, 0, 0)),
            pl.BlockSpec((H, A), lambda i: (0, 0)),
            pl.BlockSpec((1, A), lambda i: (0, 0)),
            pl.BlockSpec((1, A), lambda i: (0, 0)),
        ],
        out_specs=[
            pl.BlockSpec((TB * B, 1), lambda i: (i, 0)),
            pl.BlockSpec((TB * B, 1), lambda i: (i, 0)),
        ],
        out_shape=[
            jax.ShapeDtypeStruct((T * B, 1), jnp.float32),
            jax.ShapeDtypeStruct((T * B, 1), jnp.float32),
        ],
    )(outs3, act3, woutT.astype(jnp.bfloat16), bout, ls)

    return action, lp.reshape(T * B), ent.reshape(T * B)


def kernel(obs, hidden_states, dones, action, W_ih, W_hh, b_ih, b_hh,
           W_out, b_out, log_std):
    return _run(obs, hidden_states, dones, action, W_ih, W_hh, b_ih, b_hh,
                W_out, b_out, log_std)
